# baseline (speedup 1.0000x reference)
import functools

import jax
import jax.numpy as jnp
from jax import lax
from jax.experimental import pallas as pl
from jax.experimental.pallas import tpu as pltpu
from jax.experimental.pallas import tpu_sc as plsc

D_MODEL = 128
CHUNK = 128
NBUF = 5


@functools.cache
def _make_lookup(n_total: int, d_model: int):
    info = plsc.get_sparse_core_info()
    nc, ns = info.num_cores, info.num_subcores
    nw = nc * ns
    assert n_total % (nw * CHUNK * NBUF) == 0
    n_per_w = n_total // nw
    n_chunks = n_per_w // CHUNK
    n_rounds = n_chunks // NBUF
    mesh = plsc.VectorSubcoreMesh(core_axis_name="c", subcore_axis_name="s")

    @functools.partial(
        pl.kernel,
        mesh=mesh,
        out_type=jax.ShapeDtypeStruct((n_total, d_model), jnp.float32),
        scratch_types=[
            pltpu.VMEM((n_chunks, CHUNK), jnp.int32),
            *(pltpu.VMEM((CHUNK, d_model), jnp.float32) for _ in range(NBUF)),
            *(pltpu.SemaphoreType.DMA for _ in range(2 * NBUF)),
        ],
    )
    def lookup(table_hbm, idx_hbm, out_hbm, idx_v, *bufs_and_sems):
        rows = bufs_and_sems[:NBUF]
        gsem = bufs_and_sems[NBUF : 2 * NBUF]
        ssem = bufs_and_sems[2 * NBUF :]
        wid = lax.axis_index("s") * nc + lax.axis_index("c")
        base_w = wid * n_per_w

        pltpu.sync_copy(idx_hbm.at[pl.ds(wid * n_chunks, n_chunks)], idx_v)

        def gather(g, b):
            return pltpu.make_async_copy(table_hbm.at[idx_v.at[g]], rows[b], gsem[b])

        def store(g, b):
            return pltpu.make_async_copy(
                rows[b], out_hbm.at[pl.ds(base_w + g * CHUNK, CHUNK)], ssem[b]
            )

        for b in range(NBUF):
            gather(b, b).start()

        def body(r, carry):
            g0 = r * NBUF
            for b in range(NBUF):
                gather(g0 + b, b).wait()
                store(g0 + b, b).start()
            for b in range(NBUF):
                g_next = g0 + b + NBUF

                @pl.when(r < n_rounds - 1)
                def _():
                    store(g0 + b, b).wait()
                    gather(g_next, b).start()

            return carry

        lax.fori_loop(0, n_rounds, body, 0)

        for b in range(NBUF):
            store(n_chunks - NBUF + b, b).wait()

    return lookup


def kernel(token_ids, weight):
    b, l = token_ids.shape
    idx_2d = token_ids.reshape(-1, CHUNK).astype(jnp.int32)
    out = _make_lookup(b * l, weight.shape[1])(weight, idx_2d)
    return out.reshape(b, l, weight.shape[1])


# speedup vs baseline: 1.0051x; 1.0051x over previous
import functools

import jax
import jax.numpy as jnp
from jax import lax
from jax.experimental import pallas as pl
from jax.experimental.pallas import tpu as pltpu
from jax.experimental.pallas import tpu_sc as plsc

D_MODEL = 128
CHUNK = 128
NBUF = 5


@functools.cache
def _make_lookup(n_total: int, d_model: int):
    info = plsc.get_sparse_core_info()
    nc, ns = info.num_cores, info.num_subcores
    nw = nc * ns
    assert n_total % (nw * CHUNK * NBUF) == 0
    n_per_w = n_total // nw
    n_chunks = n_per_w // CHUNK
    n_rounds = n_chunks // NBUF
    mesh = plsc.VectorSubcoreMesh(core_axis_name="c", subcore_axis_name="s")

    @functools.partial(
        pl.kernel,
        mesh=mesh,
        out_type=jax.ShapeDtypeStruct((n_total, d_model), jnp.float32),
        scratch_types=[
            pltpu.VMEM((n_chunks, CHUNK), jnp.int32),
            *(pltpu.VMEM((CHUNK, d_model), jnp.float32) for _ in range(NBUF)),
            *(pltpu.SemaphoreType.DMA for _ in range(2 * NBUF)),
        ],
    )
    def lookup(table_hbm, idx_hbm, out_hbm, idx_v, *bufs_and_sems):
        rows = bufs_and_sems[:NBUF]
        gsem = bufs_and_sems[NBUF : 2 * NBUF]
        ssem = bufs_and_sems[2 * NBUF :]
        wid = lax.axis_index("s") * nc + lax.axis_index("c")
        base_w = wid * n_per_w

        def gather(g, b):
            return pltpu.make_async_copy(table_hbm.at[idx_v.at[g]], rows[b], gsem[b])

        def store(g, b):
            return pltpu.make_async_copy(
                rows[b], out_hbm.at[pl.ds(base_w + g * CHUNK, CHUNK)], ssem[b]
            )

        head = 8
        pltpu.sync_copy(
            idx_hbm.at[pl.ds(wid * n_chunks, head)], idx_v.at[pl.ds(0, head)]
        )
        for b in range(NBUF):
            gather(b, b).start()
        pltpu.sync_copy(
            idx_hbm.at[pl.ds(wid * n_chunks + head, n_chunks - head)],
            idx_v.at[pl.ds(head, n_chunks - head)],
        )

        def body(r, carry):
            g0 = r * NBUF
            for b in range(NBUF):
                gather(g0 + b, b).wait()
                store(g0 + b, b).start()
            for b in range(NBUF):
                g_next = g0 + b + NBUF

                @pl.when(r < n_rounds - 1)
                def _():
                    store(g0 + b, b).wait()
                    gather(g_next, b).start()

            return carry

        lax.fori_loop(0, n_rounds, body, 0)

        for b in range(NBUF):
            store(n_chunks - NBUF + b, b).wait()

    return lookup


def kernel(token_ids, weight):
    b, l = token_ids.shape
    idx_2d = token_ids.reshape(-1, CHUNK).astype(jnp.int32)
    out = _make_lookup(b * l, weight.shape[1])(weight, idx_2d)
    return out.reshape(b, l, weight.shape[1])
